# baseline (speedup 1.0000x reference)
import functools

import jax
import jax.numpy as jnp
from jax.experimental import pallas as pl
from jax.experimental.pallas import tpu as pltpu
from jax.experimental.pallas import tpu_sc as plsc

BM = 64
FT = 2048


def _glu_block_kernel(nf, be_ref, xs_ref, ss_ref, wg_ref, wu_ref, wd_ref, o_ref):
    f = pl.program_id(1)
    xb = xs_ref[...].astype(jnp.bfloat16)
    g = jnp.dot(xb, wg_ref[0].astype(jnp.bfloat16), preferred_element_type=jnp.float32)
    u = jnp.dot(xb, wu_ref[0].astype(jnp.bfloat16), preferred_element_type=jnp.float32)
    h = ((g * jax.nn.sigmoid(g)) * u).astype(jnp.bfloat16)
    acc = jnp.dot(h, wd_ref[0].astype(jnp.bfloat16), preferred_element_type=jnp.float32)

    @pl.when(f == 0)
    def _():
        o_ref[...] = acc

    @pl.when(f > 0)
    def _():
        o_ref[...] = o_ref[...] + acc

    @pl.when(f == nf - 1)
    def _():
        o_ref[...] = o_ref[...] * ss_ref[...]


def _grouped_glu(xs, ss_col, Wg, Wu, Wd, block_expert, nb, nf):
    P, D = xs.shape
    F = Wg.shape[2]
    grid_spec = pltpu.PrefetchScalarGridSpec(
        num_scalar_prefetch=1,
        grid=(nb, nf),
        in_specs=[
            pl.BlockSpec((BM, D), lambda b, f, be: (b, 0)),
            pl.BlockSpec((BM, 1), lambda b, f, be: (b, 0)),
            pl.BlockSpec((1, D, FT), lambda b, f, be: (be[b], 0, f)),
            pl.BlockSpec((1, D, FT), lambda b, f, be: (be[b], 0, f)),
            pl.BlockSpec((1, FT, D), lambda b, f, be: (be[b], f, 0)),
        ],
        out_specs=pl.BlockSpec((BM, D), lambda b, f, be: (b, 0)),
    )
    return pl.pallas_call(
        functools.partial(_glu_block_kernel, nf),
        grid_spec=grid_spec,
        out_shape=jax.ShapeDtypeStruct((P, D), jnp.float32),
        compiler_params=pltpu.CompilerParams(
            dimension_semantics=("arbitrary", "arbitrary"),
        ),
    )(block_expert, xs, ss_col, Wg, Wu, Wd)


_SC_MESH = dict(
    mesh=plsc.VectorSubcoreMesh(core_axis_name="core", subcore_axis_name="subcore"),
)


_W = 32
_NW = 32


def _sc_gather_rows(x, gidx, P):
    T, D = x.shape
    per_tile = P // _NW
    nwin = per_tile // _W
    NBUF = 3

    @functools.partial(
        pl.kernel,
        out_type=jax.ShapeDtypeStruct((P, D), x.dtype),
        scratch_types=[
            pltpu.VMEM((per_tile,), jnp.int32),
            pltpu.VMEM((NBUF, _W, D), x.dtype),
            pltpu.SemaphoreType.DMA,
            pltpu.SemaphoreType.DMA,
        ],
        **_SC_MESH,
    )
    def k(x_hbm, i_hbm, o_hbm, idxbuf, bufs, gsem, osem):
        wid = jax.lax.axis_index("subcore") * 2 + jax.lax.axis_index("core")
        base = wid * per_tile
        pltpu.sync_copy(i_hbm.at[pl.ds(base, per_tile)], idxbuf)
        gets, puts = {}, {}
        for w in range(nwin + 1):
            b = w % NBUF
            if w < nwin:
                if w >= NBUF:
                    puts[w - NBUF].wait()
                gets[w] = pltpu.async_copy(
                    x_hbm.at[idxbuf.at[pl.ds(w * _W, _W)]], bufs.at[b], gsem
                )
            if w >= 1:
                wp = w - 1
                gets[wp].wait()
                puts[wp] = pltpu.async_copy(
                    bufs.at[wp % NBUF], o_hbm.at[pl.ds(base + wp * _W, _W)], osem
                )
        for w in range(max(nwin - NBUF, 0), nwin):
            puts[w].wait()

    return k(x, gidx)


def _sc_combine(out_rows, slot_pos, T, K):
    P, D = out_rows.shape
    S = T * K
    per_tile = S // _NW
    nwin = per_tile // _W
    yw = _W // K
    NBUF = 2

    @functools.partial(
        pl.kernel,
        out_type=jax.ShapeDtypeStruct((T, D), out_rows.dtype),
        scratch_types=[
            pltpu.VMEM((per_tile,), jnp.int32),
            pltpu.VMEM((NBUF, _W, D), out_rows.dtype),
            pltpu.VMEM((NBUF, yw, D), out_rows.dtype),
            pltpu.SemaphoreType.DMA,
            pltpu.SemaphoreType.DMA,
        ],
        **_SC_MESH,
    )
    def k(rows_hbm, sp_hbm, y_hbm, idxbuf, gbufs, ybufs, gsem, osem):
        wid = jax.lax.axis_index("subcore") * 2 + jax.lax.axis_index("core")
        base = wid * per_tile
        ybase = wid * nwin * yw
        pltpu.sync_copy(sp_hbm.at[pl.ds(base, per_tile)], idxbuf)
        gets, puts = {}, {}
        for w in range(nwin + 1):
            b = w % NBUF
            if w < nwin:
                if w >= NBUF:
                    puts[w - NBUF].wait()
                gets[w] = pltpu.async_copy(
                    rows_hbm.at[idxbuf.at[pl.ds(w * _W, _W)]], gbufs.at[b], gsem
                )
            if w >= 1:
                wp = w - 1
                bp = wp % NBUF
                gets[wp].wait()

                @pl.loop(0, yw)
                def _(rr, bp=bp):
                    for j in range(D // 16):
                        sl = pl.ds(j * 16, 16)
                        ybufs[bp, rr, sl] = gbufs[bp, 2 * rr, sl] + gbufs[bp, 2 * rr + 1, sl]

                puts[wp] = pltpu.async_copy(
                    ybufs.at[bp], y_hbm.at[pl.ds(ybase + wp * yw, yw)], osem
                )
        for w in range(max(nwin - NBUF, 0), nwin):
            puts[w].wait()

    return k(out_rows, slot_pos)


def kernel(x, topK_indices, topK_scores, Wg, Wu, Wd):
    T, D = x.shape
    _, K = topK_indices.shape
    E, _, F = Wg.shape
    S = T * K
    P = S + E * BM
    NB = P // BM
    NF = F // FT

    idx = topK_indices.reshape(-1).astype(jnp.int32)
    scores = topK_scores.reshape(-1)

    counts = jnp.bincount(idx, length=E)
    sizes = ((counts + BM - 1) // BM) * BM
    ends = jnp.cumsum(sizes)
    starts = ends - sizes
    seg_begin = jnp.cumsum(counts) - counts

    order = jnp.argsort(idx, stable=True)
    sorted_e = idx[order]
    pos_sorted = (starts[sorted_e] + (jnp.arange(S) - seg_begin[sorted_e])).astype(jnp.int32)
    slot_pos = jnp.zeros((S,), jnp.int32).at[order].set(pos_sorted)
    gidx = jnp.zeros((P,), jnp.int32).at[pos_sorted].set((order // K).astype(jnp.int32))
    ss = jnp.zeros((P,), jnp.float32).at[pos_sorted].set(scores[order])
    block_expert = jnp.minimum(
        jnp.searchsorted(ends, jnp.arange(NB, dtype=jnp.int32) * BM, side="right"),
        E - 1,
    ).astype(jnp.int32)

    xs = _sc_gather_rows(x, gidx, P)
    out_rows = _grouped_glu(xs, ss[:, None], Wg, Wu, Wd, block_expert, NB, NF)
    y = _sc_combine(out_rows, slot_pos, T, K)
    return y


# speedup vs baseline: 1.4099x; 1.4099x over previous
import functools

import jax
import jax.numpy as jnp
from jax.experimental import pallas as pl
from jax.experimental.pallas import tpu as pltpu
from jax.experimental.pallas import tpu_sc as plsc

BM = 128
FT = 2048


def _glu_block_kernel(nf, be_ref, xs_ref, ss_ref, wg_ref, wu_ref, wd_ref, o_ref):
    f = pl.program_id(1)
    xb = xs_ref[...].astype(jnp.bfloat16)
    g = jnp.dot(xb, wg_ref[0].astype(jnp.bfloat16), preferred_element_type=jnp.float32)
    u = jnp.dot(xb, wu_ref[0].astype(jnp.bfloat16), preferred_element_type=jnp.float32)
    h = ((g * jax.nn.sigmoid(g)) * u).astype(jnp.bfloat16)
    acc = jnp.dot(h, wd_ref[0].astype(jnp.bfloat16), preferred_element_type=jnp.float32)

    @pl.when(f == 0)
    def _():
        o_ref[...] = acc

    @pl.when(f > 0)
    def _():
        o_ref[...] = o_ref[...] + acc

    @pl.when(f == nf - 1)
    def _():
        o_ref[...] = o_ref[...] * ss_ref[...]


def _grouped_glu(xs, ss_col, Wg, Wu, Wd, block_expert, nb, nf):
    P, D = xs.shape
    F = Wg.shape[2]
    grid_spec = pltpu.PrefetchScalarGridSpec(
        num_scalar_prefetch=1,
        grid=(nb, nf),
        in_specs=[
            pl.BlockSpec((BM, D), lambda b, f, be: (b, 0)),
            pl.BlockSpec((BM, 1), lambda b, f, be: (b, 0)),
            pl.BlockSpec((1, D, FT), lambda b, f, be: (be[b], 0, f)),
            pl.BlockSpec((1, D, FT), lambda b, f, be: (be[b], 0, f)),
            pl.BlockSpec((1, FT, D), lambda b, f, be: (be[b], f, 0)),
        ],
        out_specs=pl.BlockSpec((BM, D), lambda b, f, be: (b, 0)),
    )
    return pl.pallas_call(
        functools.partial(_glu_block_kernel, nf),
        grid_spec=grid_spec,
        out_shape=jax.ShapeDtypeStruct((P, D), jnp.float32),
        compiler_params=pltpu.CompilerParams(
            dimension_semantics=("arbitrary", "arbitrary"),
        ),
    )(block_expert, xs, ss_col, Wg, Wu, Wd)


_SC_MESH = dict(
    mesh=plsc.VectorSubcoreMesh(core_axis_name="core", subcore_axis_name="subcore"),
)


_W = 32
_NW = 32


def _sc_gather_rows(x, gidx, P):
    T, D = x.shape
    per_tile = P // _NW
    nwin = per_tile // _W
    NBUF = 3

    @functools.partial(
        pl.kernel,
        out_type=jax.ShapeDtypeStruct((P, D), x.dtype),
        scratch_types=[
            pltpu.VMEM((per_tile,), jnp.int32),
            pltpu.VMEM((NBUF, _W, D), x.dtype),
            pltpu.SemaphoreType.DMA,
            pltpu.SemaphoreType.DMA,
        ],
        **_SC_MESH,
    )
    def k(x_hbm, i_hbm, o_hbm, idxbuf, bufs, gsem, osem):
        wid = jax.lax.axis_index("subcore") * 2 + jax.lax.axis_index("core")
        base = wid * per_tile
        pltpu.sync_copy(i_hbm.at[pl.ds(base, per_tile)], idxbuf)
        gets, puts = {}, {}
        for w in range(nwin + 1):
            b = w % NBUF
            if w < nwin:
                if w >= NBUF:
                    puts[w - NBUF].wait()
                gets[w] = pltpu.async_copy(
                    x_hbm.at[idxbuf.at[pl.ds(w * _W, _W)]], bufs.at[b], gsem
                )
            if w >= 1:
                wp = w - 1
                gets[wp].wait()
                puts[wp] = pltpu.async_copy(
                    bufs.at[wp % NBUF], o_hbm.at[pl.ds(base + wp * _W, _W)], osem
                )
        for w in range(max(nwin - NBUF, 0), nwin):
            puts[w].wait()

    return k(x, gidx)


def _sc_combine(out_rows, slot_pos, T, K):
    P, D = out_rows.shape
    S = T * K
    per_tile = S // _NW
    nwin = per_tile // _W
    yw = _W // K
    NBUF = 2

    @functools.partial(
        pl.kernel,
        out_type=jax.ShapeDtypeStruct((T, D), out_rows.dtype),
        scratch_types=[
            pltpu.VMEM((per_tile,), jnp.int32),
            pltpu.VMEM((NBUF, _W, D), out_rows.dtype),
            pltpu.VMEM((NBUF, yw, D), out_rows.dtype),
            pltpu.SemaphoreType.DMA,
            pltpu.SemaphoreType.DMA,
        ],
        **_SC_MESH,
    )
    def k(rows_hbm, sp_hbm, y_hbm, idxbuf, gbufs, ybufs, gsem, osem):
        wid = jax.lax.axis_index("subcore") * 2 + jax.lax.axis_index("core")
        base = wid * per_tile
        ybase = wid * nwin * yw
        pltpu.sync_copy(sp_hbm.at[pl.ds(base, per_tile)], idxbuf)
        gets, puts = {}, {}
        for w in range(nwin + 1):
            b = w % NBUF
            if w < nwin:
                if w >= NBUF:
                    puts[w - NBUF].wait()
                gets[w] = pltpu.async_copy(
                    rows_hbm.at[idxbuf.at[pl.ds(w * _W, _W)]], gbufs.at[b], gsem
                )
            if w >= 1:
                wp = w - 1
                bp = wp % NBUF
                gets[wp].wait()

                @pl.loop(0, yw)
                def _(rr, bp=bp):
                    for j in range(D // 16):
                        sl = pl.ds(j * 16, 16)
                        ybufs[bp, rr, sl] = gbufs[bp, 2 * rr, sl] + gbufs[bp, 2 * rr + 1, sl]

                puts[wp] = pltpu.async_copy(
                    ybufs.at[bp], y_hbm.at[pl.ds(ybase + wp * yw, yw)], osem
                )
        for w in range(max(nwin - NBUF, 0), nwin):
            puts[w].wait()

    return k(out_rows, slot_pos)


def kernel(x, topK_indices, topK_scores, Wg, Wu, Wd):
    T, D = x.shape
    _, K = topK_indices.shape
    E, _, F = Wg.shape
    S = T * K
    P = S + E * BM
    NB = P // BM
    NF = F // FT

    idx = topK_indices.reshape(-1).astype(jnp.int32)
    scores = topK_scores.reshape(-1)

    counts = jnp.bincount(idx, length=E)
    sizes = ((counts + BM - 1) // BM) * BM
    ends = jnp.cumsum(sizes)
    starts = ends - sizes
    seg_begin = jnp.cumsum(counts) - counts

    order = jnp.argsort(idx, stable=True)
    sorted_e = idx[order]
    pos_sorted = (starts[sorted_e] + (jnp.arange(S) - seg_begin[sorted_e])).astype(jnp.int32)
    slot_pos = jnp.zeros((S,), jnp.int32).at[order].set(pos_sorted)
    gidx = jnp.zeros((P,), jnp.int32).at[pos_sorted].set((order // K).astype(jnp.int32))
    ss = jnp.zeros((P,), jnp.float32).at[pos_sorted].set(scores[order])
    block_expert = jnp.minimum(
        jnp.searchsorted(ends, jnp.arange(NB, dtype=jnp.int32) * BM, side="right"),
        E - 1,
    ).astype(jnp.int32)

    xs = _sc_gather_rows(x, gidx, P)
    out_rows = _grouped_glu(xs, ss[:, None], Wg, Wu, Wd, block_expert, NB, NF)
    y = _sc_combine(out_rows, slot_pos, T, K)
    return y


# speedup vs baseline: 1.4148x; 1.0035x over previous
import functools

import jax
import jax.numpy as jnp
from jax.experimental import pallas as pl
from jax.experimental.pallas import tpu as pltpu
from jax.experimental.pallas import tpu_sc as plsc

BM = 128
FT = 2048


def _glu_block_kernel(nf, be_ref, xs_ref, ss_ref, wg_ref, wu_ref, wd_ref, o_ref):
    f = pl.program_id(1)
    xb = xs_ref[...].astype(jnp.bfloat16)
    g = jnp.dot(xb, wg_ref[0].astype(jnp.bfloat16), preferred_element_type=jnp.float32)
    u = jnp.dot(xb, wu_ref[0].astype(jnp.bfloat16), preferred_element_type=jnp.float32)
    h = ((g * jax.nn.sigmoid(g)) * u).astype(jnp.bfloat16)
    acc = jnp.dot(h, wd_ref[0].astype(jnp.bfloat16), preferred_element_type=jnp.float32)

    @pl.when(f == 0)
    def _():
        o_ref[...] = acc

    @pl.when(f > 0)
    def _():
        o_ref[...] = o_ref[...] + acc

    @pl.when(f == nf - 1)
    def _():
        o_ref[...] = o_ref[...] * ss_ref[...]


def _grouped_glu(xs, ss_col, Wg, Wu, Wd, block_expert, nb, nf):
    P, D = xs.shape
    F = Wg.shape[2]
    grid_spec = pltpu.PrefetchScalarGridSpec(
        num_scalar_prefetch=1,
        grid=(nb, nf),
        in_specs=[
            pl.BlockSpec((BM, D), lambda b, f, be: (b, 0)),
            pl.BlockSpec((BM, 1), lambda b, f, be: (b, 0)),
            pl.BlockSpec((1, D, FT), lambda b, f, be: (be[b], 0, f)),
            pl.BlockSpec((1, D, FT), lambda b, f, be: (be[b], 0, f)),
            pl.BlockSpec((1, FT, D), lambda b, f, be: (be[b], f, 0)),
        ],
        out_specs=pl.BlockSpec((BM, D), lambda b, f, be: (b, 0)),
    )
    return pl.pallas_call(
        functools.partial(_glu_block_kernel, nf),
        grid_spec=grid_spec,
        out_shape=jax.ShapeDtypeStruct((P, D), jnp.float32),
        compiler_params=pltpu.CompilerParams(
            dimension_semantics=("arbitrary", "arbitrary"),
        ),
    )(block_expert, xs, ss_col, Wg, Wu, Wd)


_SC_MESH = dict(
    mesh=plsc.VectorSubcoreMesh(core_axis_name="core", subcore_axis_name="subcore"),
)


_W = 32
_NW = 32


def _sc_gather_rows(x, gidx, P):
    T, D = x.shape
    GW = 48
    per_tile = P // _NW
    nwin = per_tile // GW
    NBUF = 2

    @functools.partial(
        pl.kernel,
        out_type=jax.ShapeDtypeStruct((P, D), x.dtype),
        scratch_types=[
            pltpu.VMEM((per_tile,), jnp.int32),
            pltpu.VMEM((NBUF, GW, D), x.dtype),
            pltpu.SemaphoreType.DMA,
            pltpu.SemaphoreType.DMA,
        ],
        **_SC_MESH,
    )
    def k(x_hbm, i_hbm, o_hbm, idxbuf, bufs, gsem, osem):
        wid = jax.lax.axis_index("subcore") * 2 + jax.lax.axis_index("core")
        base = wid * per_tile
        pltpu.sync_copy(i_hbm.at[pl.ds(base, per_tile)], idxbuf)
        gets, puts = {}, {}
        for w in range(nwin + 1):
            b = w % NBUF
            if w < nwin:
                if w >= NBUF:
                    puts[w - NBUF].wait()
                gets[w] = pltpu.async_copy(
                    x_hbm.at[idxbuf.at[pl.ds(w * GW, GW)]], bufs.at[b], gsem
                )
            if w >= 1:
                wp = w - 1
                gets[wp].wait()
                puts[wp] = pltpu.async_copy(
                    bufs.at[wp % NBUF], o_hbm.at[pl.ds(base + wp * GW, GW)], osem
                )
        for w in range(max(nwin - NBUF, 0), nwin):
            puts[w].wait()

    return k(x, gidx)


def _sc_combine(out_rows, slot_pos, T, K):
    P, D = out_rows.shape
    S = T * K
    per_tile = S // _NW
    nwin = per_tile // _W
    yw = _W // K
    NBUF = 2

    @functools.partial(
        pl.kernel,
        out_type=jax.ShapeDtypeStruct((T, D), out_rows.dtype),
        scratch_types=[
            pltpu.VMEM((per_tile,), jnp.int32),
            pltpu.VMEM((NBUF, _W, D), out_rows.dtype),
            pltpu.VMEM((NBUF, yw, D), out_rows.dtype),
            pltpu.SemaphoreType.DMA,
            pltpu.SemaphoreType.DMA,
        ],
        **_SC_MESH,
    )
    def k(rows_hbm, sp_hbm, y_hbm, idxbuf, gbufs, ybufs, gsem, osem):
        wid = jax.lax.axis_index("subcore") * 2 + jax.lax.axis_index("core")
        base = wid * per_tile
        ybase = wid * nwin * yw
        pltpu.sync_copy(sp_hbm.at[pl.ds(base, per_tile)], idxbuf)
        gets, puts = {}, {}
        for w in range(nwin + 1):
            b = w % NBUF
            if w < nwin:
                if w >= NBUF:
                    puts[w - NBUF].wait()
                gets[w] = pltpu.async_copy(
                    rows_hbm.at[idxbuf.at[pl.ds(w * _W, _W)]], gbufs.at[b], gsem
                )
            if w >= 1:
                wp = w - 1
                bp = wp % NBUF
                gets[wp].wait()

                @pl.loop(0, yw)
                def _(rr, bp=bp):
                    for j in range(D // 16):
                        sl = pl.ds(j * 16, 16)
                        ybufs[bp, rr, sl] = gbufs[bp, 2 * rr, sl] + gbufs[bp, 2 * rr + 1, sl]

                puts[wp] = pltpu.async_copy(
                    ybufs.at[bp], y_hbm.at[pl.ds(ybase + wp * yw, yw)], osem
                )
        for w in range(max(nwin - NBUF, 0), nwin):
            puts[w].wait()

    return k(out_rows, slot_pos)


def kernel(x, topK_indices, topK_scores, Wg, Wu, Wd):
    T, D = x.shape
    _, K = topK_indices.shape
    E, _, F = Wg.shape
    S = T * K
    P = S + E * BM
    NB = P // BM
    NF = F // FT

    idx = topK_indices.reshape(-1).astype(jnp.int32)
    scores = topK_scores.reshape(-1)

    counts = jnp.bincount(idx, length=E)
    sizes = ((counts + BM - 1) // BM) * BM
    ends = jnp.cumsum(sizes)
    starts = ends - sizes
    seg_begin = jnp.cumsum(counts) - counts

    order = jnp.argsort(idx, stable=True)
    sorted_e = idx[order]
    pos_sorted = (starts[sorted_e] + (jnp.arange(S) - seg_begin[sorted_e])).astype(jnp.int32)
    slot_pos = jnp.zeros((S,), jnp.int32).at[order].set(pos_sorted)
    gidx = jnp.zeros((P,), jnp.int32).at[pos_sorted].set((order // K).astype(jnp.int32))
    ss = jnp.zeros((P,), jnp.float32).at[pos_sorted].set(scores[order])
    block_expert = jnp.minimum(
        jnp.searchsorted(ends, jnp.arange(NB, dtype=jnp.int32) * BM, side="right"),
        E - 1,
    ).astype(jnp.int32)

    xs = _sc_gather_rows(x, gidx, P)
    out_rows = _grouped_glu(xs, ss[:, None], Wg, Wu, Wd, block_expert, NB, NF)
    y = _sc_combine(out_rows, slot_pos, T, K)
    return y
